# Initial kernel scaffold; baseline (speedup 1.0000x reference)
import jax
import jax.numpy as jnp
from jax.experimental import pallas as pl


def kernel(img_batch, features, img_sizes, og_sizes, rpn_conv_w, rpn_conv_b, rpn_cls_w, rpn_cls_b, rpn_bbox_w, rpn_bbox_b, fc1_w, fc1_b, fc2_w, fc2_b, cls_w, cls_b, bbox_w, bbox_b):
    raise NotImplementedError("write your pallas kernel here")


# speedup vs baseline: 3.0248x; 3.0248x over previous
import functools

import jax
import jax.numpy as jnp
from jax.experimental import pallas as pl
from jax.experimental.pallas import tpu as pltpu

_A = 3
_STRIDE = 16
_PRE = 1000
_POST = 300
_DETS = 100
_NCLS = 91
_POOL = 7


def _gen_anchors(H, W):
    sizes = jnp.array([64.0, 128.0, 256.0], jnp.float32)
    cx = jnp.arange(W, dtype=jnp.float32) * _STRIDE
    cy = jnp.arange(H, dtype=jnp.float32) * _STRIDE
    CY, CX = jnp.meshgrid(cy, cx, indexing='ij')
    half = sizes / 2.0
    x1 = CX[None] - half[:, None, None]
    y1 = CY[None] - half[:, None, None]
    x2 = CX[None] + half[:, None, None]
    y2 = CY[None] + half[:, None, None]
    return jnp.stack([x1, y1, x2, y2], -1).reshape(-1, 4)


def _decode(boxes, deltas):
    w = boxes[:, 2] - boxes[:, 0]
    h = boxes[:, 3] - boxes[:, 1]
    cx = boxes[:, 0] + 0.5 * w
    cy = boxes[:, 1] + 0.5 * h
    dx, dy = deltas[:, 0], deltas[:, 1]
    dw = jnp.minimum(deltas[:, 2], jnp.log(1000.0 / 16.0))
    dh = jnp.minimum(deltas[:, 3], jnp.log(1000.0 / 16.0))
    pcx = dx * w + cx
    pcy = dy * h + cy
    pw = jnp.exp(dw) * w
    ph = jnp.exp(dh) * h
    return jnp.stack(
        [pcx - 0.5 * pw, pcy - 0.5 * ph, pcx + 0.5 * pw, pcy + 0.5 * ph], -1)


def _clip(b, hw):
    return jnp.stack([
        jnp.clip(b[:, 0], 0.0, hw[1]),
        jnp.clip(b[:, 1], 0.0, hw[0]),
        jnp.clip(b[:, 2], 0.0, hw[1]),
        jnp.clip(b[:, 3], 0.0, hw[0]),
    ], -1)


def _roi_pool(feat, boxes, H, W):
    fb = boxes / float(_STRIDE)
    g = (jnp.arange(_POOL, dtype=jnp.float32) + 0.5) / _POOL
    x = fb[:, 0:1] + g[None, :] * (fb[:, 2:3] - fb[:, 0:1])
    y = fb[:, 1:2] + g[None, :] * (fb[:, 3:4] - fb[:, 1:2])
    x = jnp.clip(x, 0.0, W - 1.001)
    y = jnp.clip(y, 0.0, H - 1.001)
    N = x.shape[0]
    X = jnp.broadcast_to(x[:, None, :], (N, _POOL, _POOL))
    Y = jnp.broadcast_to(y[:, :, None], (N, _POOL, _POOL))
    x0 = jnp.floor(X).astype(jnp.int32)
    y0 = jnp.floor(Y).astype(jnp.int32)
    x1 = jnp.minimum(x0 + 1, W - 1)
    y1 = jnp.minimum(y0 + 1, H - 1)
    wx = X - x0.astype(jnp.float32)
    wy = Y - y0.astype(jnp.float32)
    v00 = feat[:, y0, x0]
    v01 = feat[:, y0, x1]
    v10 = feat[:, y1, x0]
    v11 = feat[:, y1, x1]
    v = (v00 * (1 - wx) * (1 - wy) + v01 * wx * (1 - wy)
         + v10 * (1 - wx) * wy + v11 * wx * wy)
    return jnp.transpose(v, (1, 0, 2, 3)).reshape(N, -1)


def _mm_kern(a_ref, b_ref, o_ref):
    @pl.when(pl.program_id(0) == 0)
    def _init():
        o_ref[...] = jnp.zeros_like(o_ref)

    o_ref[...] += jnp.dot(a_ref[...], b_ref[...],
                          preferred_element_type=jnp.float32)


def _mm(a, b, bk):
    M, K = a.shape
    _, N = b.shape
    nk = K // bk
    return pl.pallas_call(
        _mm_kern,
        grid=(nk,),
        in_specs=[
            pl.BlockSpec((M, bk), lambda k: (0, k)),
            pl.BlockSpec((bk, N), lambda k: (k, 0)),
        ],
        out_specs=pl.BlockSpec((M, N), lambda k: (0, 0)),
        out_shape=jax.ShapeDtypeStruct((M, N), jnp.float32),
    )(a, b)


def _nms_kern(bc_ref, br_ref, sc_ref, keep_ref, iou_ref, *, n_iter, thresh):
    bc = bc_ref[...]
    br = br_ref[...]
    x1c, y1c, x2c, y2c = (bc[:, 0:1], bc[:, 1:2], bc[:, 2:3], bc[:, 3:4])
    x1r, y1r, x2r, y2r = (br[0:1, :], br[1:2, :], br[2:3, :], br[3:4, :])
    area_c = (x2c - x1c) * (y2c - y1c)
    area_r = (x2r - x1r) * (y2r - y1r)
    w = jnp.maximum(jnp.minimum(x2c, x2r) - jnp.maximum(x1c, x1r), 0.0)
    h = jnp.maximum(jnp.minimum(y2c, y2r) - jnp.maximum(y1c, y1r), 0.0)
    inter = w * h
    iou_ref[...] = inter / (area_c + area_r - inter + 1e-6)

    P = br.shape[1]
    idx = jax.lax.broadcasted_iota(jnp.int32, (1, P), 1)

    def body(step, s):
        m = jnp.max(s)
        i = jnp.min(jnp.where(s == m, idx, jnp.int32(2**30)))
        keep_ref[step] = i
        row = iou_ref[pl.ds(i, 1), :]
        return jnp.where(row >= thresh, -1e9, s)

    jax.lax.fori_loop(0, n_iter, body, sc_ref[...])


def _nms_pallas(boxes, scores, thresh, n_out, pad_to):
    n = boxes.shape[0]
    bc = jnp.zeros((pad_to, 4), jnp.float32).at[:n].set(boxes)
    sc = jnp.full((1, pad_to), -1e9, jnp.float32).at[0, :n].set(scores)
    keep = pl.pallas_call(
        functools.partial(_nms_kern, n_iter=n_out, thresh=thresh),
        out_shape=jax.ShapeDtypeStruct((n_out,), jnp.int32),
        out_specs=pl.BlockSpec(memory_space=pltpu.SMEM),
        scratch_shapes=[pltpu.VMEM((pad_to, pad_to), jnp.float32)],
    )(bc, bc.T, sc)
    return keep


def kernel(img_batch, features, img_sizes, og_sizes, rpn_conv_w, rpn_conv_b,
           rpn_cls_w, rpn_cls_b, rpn_bbox_w, rpn_bbox_b, fc1_w, fc1_b,
           fc2_w, fc2_b, cls_w, cls_b, bbox_w, bbox_b):
    B, C, H, W = features.shape
    HW = H * W
    anchors = _gen_anchors(H, W)

    fp = jnp.pad(features, ((0, 0), (0, 0), (1, 1), (1, 1)))
    cols = jnp.stack([fp[:, :, dy:dy + H, dx:dx + W]
                      for dy in range(3) for dx in range(3)], 1)
    amat = cols.reshape(B, 9 * C, HW).transpose(0, 2, 1).reshape(B * HW, 9 * C)
    m_pad = ((B * HW + 7) // 8) * 8
    amat = jnp.pad(amat, ((0, m_pad - B * HW), (0, 0)))
    wmat = rpn_conv_w.transpose(2, 3, 1, 0).reshape(9 * C, C)
    t = _mm(amat, wmat, 768)
    t = jax.nn.relu(t + rpn_conv_b[None, :])

    whead = jnp.concatenate([rpn_cls_w.T, rpn_bbox_w.T], 1)
    whead = jnp.pad(whead, ((0, 0), (0, 128 - whead.shape[1])))
    ho = _mm(t, whead, 256)[:B * HW, :5 * _A]
    obj_all = ho[:, :_A] + rpn_cls_b[None, :]
    del_all = ho[:, _A:] + rpn_bbox_b[None, :]

    pboxes_l, pooled_l, hw_l = [], [], []
    for i in range(B):
        obj = obj_all[i * HW:(i + 1) * HW].T.reshape(-1)
        deltas = (del_all[i * HW:(i + 1) * HW]
                  .reshape(HW, _A, 4).transpose(1, 0, 2).reshape(-1, 4))
        sc, idx = jax.lax.top_k(jax.nn.sigmoid(obj), _PRE)
        props = _decode(anchors[idx], deltas[idx])
        hw = img_sizes[i].astype(jnp.float32)
        props = _clip(props, hw)
        keep = _nms_pallas(props, sc, 0.7, _POST, 1024)
        pb = props[keep]
        pboxes_l.append(pb)
        pooled_l.append(_roi_pool(features[i], pb, H, W))
        hw_l.append(hw)

    pooled = jnp.concatenate(pooled_l, 0)
    p_rows = pooled.shape[0]
    p_pad = ((p_rows + 7) // 8) * 8
    pooled = jnp.pad(pooled, ((0, p_pad - p_rows), (0, 0)))
    h1 = jax.nn.relu(_mm(pooled, fc1_w, 1792) + fc1_b[None, :])
    h2 = jax.nn.relu(_mm(h1, fc2_w, 1024) + fc2_b[None, :])
    wout = jnp.concatenate([cls_w, bbox_w], 1)
    wout = jnp.pad(wout, ((0, 0), (0, 512 - wout.shape[1])))
    ho2 = _mm(h2, wout, 1024)[:p_rows]
    logits = ho2[:, :_NCLS] + cls_b[None, :]
    bdel_all = (ho2[:, _NCLS:5 * _NCLS] + bbox_b[None, :]).reshape(
        p_rows, _NCLS, 4)
    probs = jax.nn.softmax(logits, -1)

    outs = []
    for i in range(B):
        sl = slice(i * _POST, (i + 1) * _POST)
        fg = probs[sl, 1:]
        score = jnp.max(fg, -1)
        label = jnp.argmax(fg, -1) + 1
        d = bdel_all[sl][jnp.arange(_POST), label]
        fboxes = _clip(_decode(pboxes_l[i], d), hw_l[i])
        keep2 = _nms_pallas(fboxes, score, 0.5, _DETS, 384)
        fb = fboxes[keep2]
        fs = score[keep2]
        ratio = og_sizes[i].astype(jnp.float32) / hw_l[i]
        scale = jnp.stack([ratio[1], ratio[0], ratio[1], ratio[0]])
        outs.append(jnp.concatenate([fb * scale[None, :], fs[:, None]], -1))
    return jnp.stack(outs)


# speedup vs baseline: 3.2222x; 1.0652x over previous
import functools

import jax
import jax.numpy as jnp
from jax.experimental import pallas as pl
from jax.experimental.pallas import tpu as pltpu

_A = 3
_STRIDE = 16
_PRE = 1000
_POST = 300
_DETS = 100
_NCLS = 91
_POOL = 7


def _gen_anchors(H, W):
    sizes = jnp.array([64.0, 128.0, 256.0], jnp.float32)
    cx = jnp.arange(W, dtype=jnp.float32) * _STRIDE
    cy = jnp.arange(H, dtype=jnp.float32) * _STRIDE
    CY, CX = jnp.meshgrid(cy, cx, indexing='ij')
    half = sizes / 2.0
    x1 = CX[None] - half[:, None, None]
    y1 = CY[None] - half[:, None, None]
    x2 = CX[None] + half[:, None, None]
    y2 = CY[None] + half[:, None, None]
    return jnp.stack([x1, y1, x2, y2], -1).reshape(-1, 4)


def _decode(boxes, deltas):
    w = boxes[:, 2] - boxes[:, 0]
    h = boxes[:, 3] - boxes[:, 1]
    cx = boxes[:, 0] + 0.5 * w
    cy = boxes[:, 1] + 0.5 * h
    dx, dy = deltas[:, 0], deltas[:, 1]
    dw = jnp.minimum(deltas[:, 2], jnp.log(1000.0 / 16.0))
    dh = jnp.minimum(deltas[:, 3], jnp.log(1000.0 / 16.0))
    pcx = dx * w + cx
    pcy = dy * h + cy
    pw = jnp.exp(dw) * w
    ph = jnp.exp(dh) * h
    return jnp.stack(
        [pcx - 0.5 * pw, pcy - 0.5 * ph, pcx + 0.5 * pw, pcy + 0.5 * ph], -1)


def _clip(b, hw):
    return jnp.stack([
        jnp.clip(b[:, 0], 0.0, hw[1]),
        jnp.clip(b[:, 1], 0.0, hw[0]),
        jnp.clip(b[:, 2], 0.0, hw[1]),
        jnp.clip(b[:, 3], 0.0, hw[0]),
    ], -1)


def _roi_pool(feat, boxes, H, W):
    fb = boxes / float(_STRIDE)
    g = (jnp.arange(_POOL, dtype=jnp.float32) + 0.5) / _POOL
    x = fb[:, 0:1] + g[None, :] * (fb[:, 2:3] - fb[:, 0:1])
    y = fb[:, 1:2] + g[None, :] * (fb[:, 3:4] - fb[:, 1:2])
    x = jnp.clip(x, 0.0, W - 1.001)
    y = jnp.clip(y, 0.0, H - 1.001)
    N = x.shape[0]
    X = jnp.broadcast_to(x[:, None, :], (N, _POOL, _POOL))
    Y = jnp.broadcast_to(y[:, :, None], (N, _POOL, _POOL))
    x0 = jnp.floor(X).astype(jnp.int32)
    y0 = jnp.floor(Y).astype(jnp.int32)
    x1 = jnp.minimum(x0 + 1, W - 1)
    y1 = jnp.minimum(y0 + 1, H - 1)
    wx = X - x0.astype(jnp.float32)
    wy = Y - y0.astype(jnp.float32)
    v00 = feat[:, y0, x0]
    v01 = feat[:, y0, x1]
    v10 = feat[:, y1, x0]
    v11 = feat[:, y1, x1]
    v = (v00 * (1 - wx) * (1 - wy) + v01 * wx * (1 - wy)
         + v10 * (1 - wx) * wy + v11 * wx * wy)
    return jnp.transpose(v, (1, 0, 2, 3)).reshape(N, -1)


def _mm_kern(a_ref, b_ref, o_ref):
    @pl.when(pl.program_id(0) == 0)
    def _init():
        o_ref[...] = jnp.zeros_like(o_ref)

    o_ref[...] += jnp.dot(a_ref[...], b_ref[...],
                          preferred_element_type=jnp.float32)


def _mm(a, b, bk):
    M, K = a.shape
    _, N = b.shape
    nk = K // bk
    return pl.pallas_call(
        _mm_kern,
        grid=(nk,),
        in_specs=[
            pl.BlockSpec((M, bk), lambda k: (0, k)),
            pl.BlockSpec((bk, N), lambda k: (k, 0)),
        ],
        out_specs=pl.BlockSpec((M, N), lambda k: (0, 0)),
        out_shape=jax.ShapeDtypeStruct((M, N), jnp.float32),
    )(a, b)


def _nms_kern(bc_ref, br_ref, sc_ref, keep_ref, iou_ref, *, n_iter, thresh):
    bc = bc_ref[...]
    br = br_ref[...]
    x1c, y1c, x2c, y2c = (bc[:, 0:1], bc[:, 1:2], bc[:, 2:3], bc[:, 3:4])
    x1r, y1r, x2r, y2r = (br[0:1, :], br[1:2, :], br[2:3, :], br[3:4, :])
    area_c = (x2c - x1c) * (y2c - y1c)
    area_r = (x2r - x1r) * (y2r - y1r)
    w = jnp.maximum(jnp.minimum(x2c, x2r) - jnp.maximum(x1c, x1r), 0.0)
    h = jnp.maximum(jnp.minimum(y2c, y2r) - jnp.maximum(y1c, y1r), 0.0)
    inter = w * h
    iou_ref[...] = inter / (area_c + area_r - inter + 1e-6)

    P = br.shape[1]
    idx = jax.lax.broadcasted_iota(jnp.int32, (1, P), 1)

    def body(step, s):
        m = jnp.max(s)
        i = jnp.min(jnp.where(s == m, idx, jnp.int32(2**30)))
        keep_ref[step] = i
        row = iou_ref[pl.ds(i, 1), :]
        return jnp.where(row >= thresh, -1e9, s)

    jax.lax.fori_loop(0, n_iter, body, sc_ref[...])


def _nms_pallas(boxes, scores, thresh, n_out, pad_to):
    n = boxes.shape[0]
    bc = jnp.zeros((pad_to, 4), jnp.float32).at[:n].set(boxes)
    sc = jnp.full((1, pad_to), -1e9, jnp.float32).at[0, :n].set(scores)
    keep = pl.pallas_call(
        functools.partial(_nms_kern, n_iter=n_out, thresh=thresh),
        out_shape=jax.ShapeDtypeStruct((n_out,), jnp.int32),
        out_specs=pl.BlockSpec(memory_space=pltpu.SMEM),
        scratch_shapes=[pltpu.VMEM((pad_to, pad_to), jnp.float32)],
    )(bc, bc.T, sc)
    return keep


def kernel(img_batch, features, img_sizes, og_sizes, rpn_conv_w, rpn_conv_b,
           rpn_cls_w, rpn_cls_b, rpn_bbox_w, rpn_bbox_b, fc1_w, fc1_b,
           fc2_w, fc2_b, cls_w, cls_b, bbox_w, bbox_b):
    B, C, H, W = features.shape
    HW = H * W
    anchors = _gen_anchors(H, W)

    fpt = jnp.pad(features.transpose(0, 2, 3, 1),
                  ((0, 0), (1, 1), (1, 1), (0, 0)))
    cols = jnp.concatenate([fpt[:, dy:dy + H, dx:dx + W, :].reshape(B, HW, C)
                            for dy in range(3) for dx in range(3)], -1)
    amat = cols.reshape(B * HW, 9 * C)
    m_pad = ((B * HW + 7) // 8) * 8
    amat = jnp.pad(amat, ((0, m_pad - B * HW), (0, 0)))
    wmat = rpn_conv_w.transpose(2, 3, 1, 0).reshape(9 * C, C)
    t = _mm(amat, wmat, 768)
    t = jax.nn.relu(t + rpn_conv_b[None, :])

    whead = jnp.concatenate([rpn_cls_w.T, rpn_bbox_w.T], 1)
    whead = jnp.pad(whead, ((0, 0), (0, 128 - whead.shape[1])))
    ho = _mm(t, whead, 256)[:B * HW, :5 * _A]
    obj_all = ho[:, :_A] + rpn_cls_b[None, :]
    del_all = ho[:, _A:] + rpn_bbox_b[None, :]

    pboxes_l, pooled_l, hw_l = [], [], []
    for i in range(B):
        obj = obj_all[i * HW:(i + 1) * HW].T.reshape(-1)
        deltas = (del_all[i * HW:(i + 1) * HW]
                  .reshape(HW, _A, 4).transpose(1, 0, 2).reshape(-1, 4))
        sc, idx = jax.lax.top_k(jax.nn.sigmoid(obj), _PRE)
        props = _decode(anchors[idx], deltas[idx])
        hw = img_sizes[i].astype(jnp.float32)
        props = _clip(props, hw)
        keep = _nms_pallas(props, sc, 0.7, _POST, 1024)
        pb = props[keep]
        pboxes_l.append(pb)
        pooled_l.append(_roi_pool(features[i], pb, H, W))
        hw_l.append(hw)

    pooled = jnp.concatenate(pooled_l, 0)
    p_rows = pooled.shape[0]
    p_pad = ((p_rows + 7) // 8) * 8
    pooled = jnp.pad(pooled, ((0, p_pad - p_rows), (0, 0)))
    h1 = jax.nn.relu(_mm(pooled, fc1_w, 1792) + fc1_b[None, :])
    h2 = jax.nn.relu(_mm(h1, fc2_w, 1024) + fc2_b[None, :])
    wout = jnp.concatenate([cls_w, bbox_w], 1)
    wout = jnp.pad(wout, ((0, 0), (0, 512 - wout.shape[1])))
    ho2 = _mm(h2, wout, 1024)[:p_rows]
    logits = ho2[:, :_NCLS] + cls_b[None, :]
    bdel_all = (ho2[:, _NCLS:5 * _NCLS] + bbox_b[None, :]).reshape(
        p_rows, _NCLS, 4)
    probs = jax.nn.softmax(logits, -1)

    outs = []
    for i in range(B):
        sl = slice(i * _POST, (i + 1) * _POST)
        fg = probs[sl, 1:]
        score = jnp.max(fg, -1)
        label = jnp.argmax(fg, -1) + 1
        d = bdel_all[sl][jnp.arange(_POST), label]
        fboxes = _clip(_decode(pboxes_l[i], d), hw_l[i])
        keep2 = _nms_pallas(fboxes, score, 0.5, _DETS, 384)
        fb = fboxes[keep2]
        fs = score[keep2]
        ratio = og_sizes[i].astype(jnp.float32) / hw_l[i]
        scale = jnp.stack([ratio[1], ratio[0], ratio[1], ratio[0]])
        outs.append(jnp.concatenate([fb * scale[None, :], fs[:, None]], -1))
    return jnp.stack(outs)
